# baseline (speedup 1.0000x reference)
import jax
import jax.numpy as jnp
import numpy as np
from jax.experimental import pallas as pl

N = 10000
HID = 128
REL = 2
HEADS = 8
INV_W = 0.1


def _lrelu(x):
    return jax.nn.leaky_relu(x, 0.01)


def _layernorm(x, g, b):
    mu = x.mean(-1, keepdims=True)
    var = ((x - mu) ** 2).mean(-1, keepdims=True)
    return (x - mu) / jnp.sqrt(var + 1e-5) * g + b


def _mha(x, p):
    n, L, D = x.shape
    hd = D // HEADS
    q = (x @ p['wq'] + p['bq']).reshape(n, L, HEADS, hd).transpose(0, 2, 1, 3)
    k = (x @ p['wk'] + p['bk']).reshape(n, L, HEADS, hd).transpose(0, 2, 1, 3)
    v = (x @ p['wv'] + p['bv']).reshape(n, L, HEADS, hd).transpose(0, 2, 1, 3)
    s = jnp.einsum('nhld,nhmd->nhlm', q, k) / np.sqrt(hd)
    a = jax.nn.softmax(s, axis=-1)
    o = jnp.einsum('nhlm,nhmd->nhld', a, v).transpose(0, 2, 1, 3).reshape(n, L, D)
    return o @ p['wo'] + p['bo']


def _seg_softmax(logits, seg, n):
    m = jax.ops.segment_max(logits, seg, num_segments=n)
    m = jnp.where(jnp.isfinite(m), m, 0.0)
    ex = jnp.exp(logits - m[seg])
    s = jax.ops.segment_sum(ex, seg, num_segments=n)
    return ex / (s[seg] + 1e-16)


def _tconv(x, edge_index, edge_attr, p):
    n = x.shape[0]
    C = HID // HEADS
    src = edge_index[0]
    dst = edge_index[1]
    q = (x @ p['wq'] + p['bq']).reshape(n, HEADS, C)
    k = (x @ p['wk'] + p['bk']).reshape(n, HEADS, C)
    v = (x @ p['wv'] + p['bv']).reshape(n, HEADS, C)
    e = (edge_attr @ p['we']).reshape(-1, HEADS, C)
    kj = k[src] + e
    vj = v[src] + e
    alpha = (q[dst] * kj).sum(-1) / np.sqrt(C)
    alpha = _seg_softmax(alpha, dst, n)
    out = jax.ops.segment_sum(alpha[:, :, None] * vj, dst, num_segments=n).reshape(n, HID)
    return out + x @ p['wskip'] + p['bskip']


def _cos(a, b):
    na = jnp.maximum(jnp.linalg.norm(a, axis=1), 1e-8)
    nb = jnp.maximum(jnp.linalg.norm(b, axis=1), 1e-8)
    return (a * b).sum(1) / (na * nb)


def _touch_kernel(x_ref, o_ref):
    o_ref[...] = x_ref[...]


def kernel(description, tweet, num_prop, cat_prop, edge_index, edge_type, params):
    mods = []
    for t, nm in ((description, 'desc'), (tweet, 'tweet'), (num_prop, 'num'), (cat_prop, 'cat')):
        p = params[nm]
        mods.append(_lrelu(t @ p['w'] + p['b']))
    inv_parts, spec_parts, chans = [], [], []
    for i, m in enumerate(mods):
        pi = params['inv'][i]
        ps = params['spec'][i]
        inv = jnp.tanh(m @ pi['w'] + pi['b'])
        spec = _lrelu(m @ ps['w'] + ps['b'])
        inv_parts.append(inv)
        spec_parts.append(spec)
        chans.append(jnp.concatenate((inv, spec), axis=1))
    ct = jnp.stack(chans, axis=1)
    att = _mha(ct, params['mha'])
    ct = _layernorm(att + ct, params['ln_g'], params['ln_b'])
    fused = _lrelu(ct.mean(axis=1) @ params['c2h']['w'] + params['c2h']['b'])
    et = jnp.clip(edge_type, 0, REL - 1)
    edge_attr = params['rel_emb'][et]
    x = _tconv(fused, edge_index, edge_attr, params['conv1'])
    x = _lrelu(x)
    x = _tconv(x, edge_index, edge_attr, params['conv2'])
    x = _lrelu(x @ params['outmlp']['w'] + params['outmlp']['b'])
    logits = x @ params['head']['w'] + params['head']['b']
    inv_stack = jnp.stack(inv_parts, axis=1)
    center = inv_stack.mean(axis=1, keepdims=True)
    inv_loss = ((inv_stack - center) ** 2).mean()
    overlap = jnp.float32(0.0)
    cnt = 0
    for l in range(4):
        for r in range(l + 1, 4):
            overlap = overlap + jnp.abs(_cos(spec_parts[l], spec_parts[r])).mean()
            cnt += 1
    overlap = overlap / cnt
    aux = INV_W * (inv_loss + 0.5 * overlap)
    logits = pl.pallas_call(
        _touch_kernel,
        out_shape=jax.ShapeDtypeStruct(logits.shape, logits.dtype),
    )(logits)
    return logits, aux


# speedup vs baseline: 36.3337x; 36.3337x over previous
import functools

import jax
import jax.numpy as jnp
import numpy as np
from jax import lax
from jax.experimental import pallas as pl
from jax.experimental.pallas import tpu as pltpu
from jax.experimental.pallas import tpu_sc as plsc

N = 10000
E = 320000
HID = 128
HEADS = 8
C = HID // HEADS
REL = 2
INV_W = 0.1
F32 = jnp.float32

NC = 2
NS = 16
NW = NC * NS
NPAD = 10112
DRP = 2 * NPAD
DR8 = DRP // 8
DR8P = 2560
CHK = 32
EPT = 10112
EP = EPT * NW
NCHUNK = EPT // CHK
ROWS_A = NPAD // NS
ROWS_S = DR8P // NS

PERM = np.array([(f % HEADS) * C + f // HEADS for f in range(HID)])

def _make_mesh():
    return plsc.VectorSubcoreMesh(core_axis_name="c", subcore_axis_name="s",
                                  num_cores=NC, num_subcores=NS)


def _sc_edge_body(src_h, dst_h, et_h, qt_h, kt_h, vt_h, qe_h,
                  acc_o, sacc_o,
                  srcb, dstb, etb, drb, dr8b, qrows, krows, vrows, wbuf, w2,
                  qerows,
                  acc_sh, sacc_sh, sem1, sem2, sem3, sem4):
    cid = lax.axis_index("c")
    sid = lax.axis_index("s")
    wid = cid * NS + sid

    zero16 = jnp.zeros((16,), F32)

    def zrow(i, _):
        for g in range(HID // 16):
            wbuf[i, pl.ds(g * 16, 16)] = zero16
        return 0

    lax.fori_loop(0, CHK, zrow, 0)

    za = sid * ROWS_A
    for t in range((ROWS_A + CHK - 1) // CHK):
        nrows = min(CHK, ROWS_A - t * CHK)
        pltpu.sync_copy(wbuf.at[pl.ds(0, nrows)],
                        acc_sh.at[pl.ds(za + t * CHK, nrows)])
    zs = sid * ROWS_S
    for t in range((ROWS_S + CHK - 1) // CHK):
        nrows = min(CHK, ROWS_S - t * CHK)
        pltpu.sync_copy(wbuf.at[pl.ds(0, nrows)],
                        sacc_sh.at[pl.ds(zs + t * CHK, nrows)])
    plsc.subcore_barrier()

    swp = lax.iota(jnp.int32, 16) ^ 8
    ebase = wid * EPT

    def chunk(g, _):
        off = ebase + g * CHK
        pltpu.sync_copy(src_h.at[pl.ds(off, CHK)], srcb)
        pltpu.sync_copy(dst_h.at[pl.ds(off, CHK)], dstb)
        pltpu.sync_copy(et_h.at[pl.ds(off, CHK)], etb)
        for j in range(CHK // 16):
            dv = dstb[pl.ds(j * 16, 16)]
            rv = etb[pl.ds(j * 16, 16)]
            rv = jnp.minimum(jnp.maximum(rv, 0), REL - 1)
            dr = rv * NPAD + dv
            drb[pl.ds(j * 16, 16)] = dr
            dr8b[pl.ds(j * 16, 16)] = lax.shift_right_logical(dr, 3)
        cp1 = pltpu.async_copy(qt_h.at[dstb], qrows, sem1)
        cp2 = pltpu.async_copy(kt_h.at[srcb], krows, sem2)
        cp3 = pltpu.async_copy(vt_h.at[srcb], vrows, sem3)
        cp4 = pltpu.async_copy(qe_h.at[drb], qerows, sem4)
        cp1.wait()
        cp2.wait()
        cp3.wait()
        cp4.wait()

        def edge(j, _):
            t = qrows[j, pl.ds(0, 16)] * krows[j, pl.ds(0, 16)]
            for g2 in range(1, HID // 16):
                t = t + qrows[j, pl.ds(g2 * 16, 16)] * krows[j, pl.ds(g2 * 16, 16)]
            u = t + lax.gather(
                t, swp[:, None],
                lax.GatherDimensionNumbers(offset_dims=(), collapsed_slice_dims=(0,),
                                           start_index_map=(0,)),
                (1,), mode=lax.GatherScatterMode.PROMISE_IN_BOUNDS)
            ex = jnp.exp(u + qerows[j, pl.ds(0, 16)])
            slotf = qerows[j, pl.ds(16, 16)]
            for g2 in range(HID // 16):
                wbuf[j, pl.ds(g2 * 16, 16)] = vrows[j, pl.ds(g2 * 16, 16)] * ex
                w2[j, pl.ds(g2 * 16, 16)] = jnp.where(slotf == float(g2), ex, zero16)
            return 0

        lax.fori_loop(0, CHK, edge, 0)
        pltpu.sync_copy(w2, sacc_sh.at[dr8b], add=True)
        pltpu.sync_copy(wbuf, acc_sh.at[dstb], add=True)
        return 0

    lax.fori_loop(0, NCHUNK, chunk, 0)
    plsc.subcore_barrier()

    oa = cid * NPAD + sid * ROWS_A
    for t in range((ROWS_A + CHK - 1) // CHK):
        nrows = min(CHK, ROWS_A - t * CHK)
        pltpu.sync_copy(acc_sh.at[pl.ds(sid * ROWS_A + t * CHK, nrows)],
                        acc_o.at[pl.ds(oa + t * CHK, nrows)])
    os_ = cid * DR8P + sid * ROWS_S
    for t in range((ROWS_S + CHK - 1) // CHK):
        nrows = min(CHK, ROWS_S - t * CHK)
        pltpu.sync_copy(sacc_sh.at[pl.ds(sid * ROWS_S + t * CHK, nrows)],
                        sacc_o.at[pl.ds(os_ + t * CHK, nrows)])


_SC_EDGE_CACHE = []


def _sc_edge(*args):
    if not _SC_EDGE_CACHE:
        _SC_EDGE_CACHE.append(_build_sc_edge())
    return _SC_EDGE_CACHE[0](*args)


def _build_sc_edge():
    return functools.partial(
        pl.kernel,
        out_type=(jax.ShapeDtypeStruct((NC * NPAD, HID), F32),
                  jax.ShapeDtypeStruct((NC * DR8P, HID), F32)),
        mesh=_make_mesh(),
        scratch_types=[
            pltpu.VMEM((CHK,), jnp.int32),
            pltpu.VMEM((CHK,), jnp.int32),
            pltpu.VMEM((CHK,), jnp.int32),
            pltpu.VMEM((CHK,), jnp.int32),
            pltpu.VMEM((CHK,), jnp.int32),
            pltpu.VMEM((CHK, HID), F32),
            pltpu.VMEM((CHK, HID), F32),
            pltpu.VMEM((CHK, HID), F32),
            pltpu.VMEM((CHK, HID), F32),
            pltpu.VMEM((CHK, HID), F32),
            pltpu.VMEM((CHK, HID), F32),
            pltpu.VMEM_SHARED((NPAD, HID), F32),
            pltpu.VMEM_SHARED((DR8P, HID), F32),
            pltpu.SemaphoreType.DMA,
            pltpu.SemaphoreType.DMA,
            pltpu.SemaphoreType.DMA,
            pltpu.SemaphoreType.DMA,
        ],
    )(_sc_edge_body)


B = 1000
GRID = N // B

LRELU = 0.01


def _lrelu(x):
    return jnp.where(x > 0, x, x * LRELU)


def _head_mask64():
    r = lax.broadcasted_iota(jnp.int32, (64, 8), 0)
    c = lax.broadcasted_iota(jnp.int32, (64, 8), 1)
    return (r // 8 == c).astype(F32)


def _head_mask128():
    r = lax.broadcasted_iota(jnp.int32, (HID, 8), 0)
    c = lax.broadcasted_iota(jnp.int32, (HID, 8), 1)
    return (r % 8 == c).astype(F32)


def _bcast16():
    r = lax.broadcasted_iota(jnp.int32, (16, HID), 0)
    c = lax.broadcasted_iota(jnp.int32, (16, HID), 1)
    return (r % 8 == c % 8).astype(F32) * 0.5


def _tc1_body(desc_ref, tw_ref, np_ref, cp_ref,
              wd, bd, wt, bt, wn, bn, wc, bc,
              winv, binv, wspec, bspec,
              wqm, bqm, wkm, bkm, wvm, bvm, wom, bom, lng, lnb,
              c2hw, c2hb,
              wq1, bq1, wk1, bk1, wv1, bv1, ws1, bs1, ek1,
              q1_o, k1_o, v1_o, s1_o, qe0_o, qe1_o, aux_o):
    mods = [
        _lrelu(jnp.dot(desc_ref[...], wd[...], preferred_element_type=F32) + bd[...]),
        _lrelu(jnp.dot(tw_ref[...], wt[...], preferred_element_type=F32) + bt[...]),
        _lrelu(jnp.dot(np_ref[...], wn[...], preferred_element_type=F32) + bn[...]),
        _lrelu(jnp.dot(cp_ref[...], wc[...], preferred_element_type=F32) + bc[...]),
    ]
    invs, specs, toks = [], [], []
    for i in range(4):
        inv = jnp.tanh(jnp.dot(mods[i], winv[i], preferred_element_type=F32) + binv[i])
        spec = _lrelu(jnp.dot(mods[i], wspec[i], preferred_element_type=F32) + bspec[i])
        invs.append(inv)
        specs.append(spec)
        toks.append(jnp.concatenate((inv, spec), axis=1))
    mh = _head_mask64()
    qs = [jnp.dot(t, wqm[...], preferred_element_type=F32) + bqm[...] for t in toks]
    ks = [jnp.dot(t, wkm[...], preferred_element_type=F32) + bkm[...] for t in toks]
    vs = [jnp.dot(t, wvm[...], preferred_element_type=F32) + bvm[...] for t in toks]
    scale = 1.0 / np.sqrt(8.0)
    ct_out = []
    for l in range(4):
        s_lm = [jnp.dot(qs[l] * ks[m], mh, preferred_element_type=F32) * scale
                for m in range(4)]
        mx = jnp.maximum(jnp.maximum(s_lm[0], s_lm[1]),
                         jnp.maximum(s_lm[2], s_lm[3]))
        e_lm = [jnp.exp(s - mx) for s in s_lm]
        ssum = e_lm[0] + e_lm[1] + e_lm[2] + e_lm[3]
        o_l = 0.0
        for m in range(4):
            a = e_lm[m] / ssum
            o_l = o_l + jnp.dot(a, mh.T, preferred_element_type=F32) * vs[m]
        att = jnp.dot(o_l, wom[...], preferred_element_type=F32) + bom[...]
        x = att + toks[l]
        mu = jnp.mean(x, axis=1, keepdims=True)
        var = jnp.mean((x - mu) ** 2, axis=1, keepdims=True)
        ct_out.append((x - mu) / jnp.sqrt(var + 1e-5) * lng[...] + lnb[...])
    cmean = (ct_out[0] + ct_out[1] + ct_out[2] + ct_out[3]) * 0.25
    fused = _lrelu(jnp.dot(cmean, c2hw[...], preferred_element_type=F32) + c2hb[...])

    q1 = jnp.dot(fused, wq1[...], preferred_element_type=F32) + bq1[...]
    k1 = jnp.dot(fused, wk1[...], preferred_element_type=F32) + bk1[...]
    v1 = jnp.dot(fused, wv1[...], preferred_element_type=F32) + bv1[...]
    s1 = jnp.dot(fused, ws1[...], preferred_element_type=F32) + bs1[...]
    q1_o[...] = q1
    k1_o[...] = k1
    v1_o[...] = v1
    s1_o[...] = s1
    mh128 = _head_mask128()
    nb = q1.shape[0]
    zpad = jnp.zeros((nb, HID - 32), F32)
    d_id = pl.program_id(0) * B + lax.broadcasted_iota(jnp.int32, (nb, 16), 0)
    for r, ref in ((0, qe0_o), (1, qe1_o)):
        qe8 = jnp.dot(q1 * ek1[r], mh128, preferred_element_type=F32)
        slotf = ((r * NPAD + d_id) % 8).astype(F32)
        ref[...] = jnp.concatenate((qe8, qe8, slotf, zpad), axis=1)

    center = (invs[0] + invs[1] + invs[2] + invs[3]) * 0.25
    inv_ss = 0.0
    for i in range(4):
        d = invs[i] - center
        inv_ss = inv_ss + jnp.sum(d * d)
    nrm = [jnp.maximum(jnp.sqrt(jnp.sum(s * s, axis=1, keepdims=True)), 1e-8)
           for s in specs]
    ov_ss = 0.0
    for l in range(4):
        for r in range(l + 1, 4):
            dot = jnp.sum(specs[l] * specs[r], axis=1, keepdims=True)
            ov_ss = ov_ss + jnp.sum(jnp.abs(dot / (nrm[l] * nrm[r])))
    lane = lax.broadcasted_iota(jnp.int32, (1, HID), 1)
    vec = jnp.where(lane == 0, inv_ss, jnp.where(lane == 1, ov_ss, 0.0))

    @pl.when(pl.program_id(0) == 0)
    def _():
        aux_o[...] = vec

    @pl.when(pl.program_id(0) != 0)
    def _():
        aux_o[...] = aux_o[...] + vec


def _combine(accs, saccs, ev, skip):
    bc = _bcast16()
    acc = accs[0] + accs[1]
    s0 = saccs[0, 0] + saccs[1, 0]
    s1 = saccs[0, 1] + saccs[1, 1]
    b0 = jnp.dot(s0, bc, preferred_element_type=F32)
    b1 = jnp.dot(s1, bc, preferred_element_type=F32)
    term = b0 * ev[0] + b1 * ev[1]
    denom = b0 + b1 + 1e-16
    return (acc + term) / denom + skip


def _tc2_body(accs_ref, saccs_ref, skip1_ref,
              ev1, wq2, bq2, wk2, bk2, wv2, bv2, ws2, bs2, ek2,
              q2_o, k2_o, v2_o, s2_o, qe0_o, qe1_o):
    out = _combine(accs_ref[...], saccs_ref[...], ev1[...], skip1_ref[...])
    x1 = _lrelu(out)
    q2 = jnp.dot(x1, wq2[...], preferred_element_type=F32) + bq2[...]
    k2 = jnp.dot(x1, wk2[...], preferred_element_type=F32) + bk2[...]
    v2 = jnp.dot(x1, wv2[...], preferred_element_type=F32) + bv2[...]
    s2 = jnp.dot(x1, ws2[...], preferred_element_type=F32) + bs2[...]
    q2_o[...] = q2
    k2_o[...] = k2
    v2_o[...] = v2
    s2_o[...] = s2
    mh128 = _head_mask128()
    nb = q2.shape[0]
    zpad = jnp.zeros((nb, HID - 32), F32)
    d_id = pl.program_id(0) * B + lax.broadcasted_iota(jnp.int32, (nb, 16), 0)
    for r, ref in ((0, qe0_o), (1, qe1_o)):
        qe8 = jnp.dot(q2 * ek2[r], mh128, preferred_element_type=F32)
        slotf = ((r * NPAD + d_id) % 8).astype(F32)
        ref[...] = jnp.concatenate((qe8, qe8, slotf, zpad), axis=1)


def _tc3_body(accs_ref, saccs_ref, skip2_ref,
              ev2, womlp, bomlp, whead, bhead,
              lp_o):
    out = _combine(accs_ref[...], saccs_ref[...], ev2[...], skip2_ref[...])
    y = _lrelu(jnp.dot(out, womlp[...], preferred_element_type=F32) + bomlp[...])
    lp_o[...] = jnp.dot(y, whead[...], preferred_element_type=F32) + bhead[...]


def _full(shape):
    nd = len(shape)
    return pl.BlockSpec(shape, lambda i: (0,) * nd)


def _rows(shape):
    nd = len(shape)
    return pl.BlockSpec(shape, lambda i, _nd=nd: (i,) + (0,) * (_nd - 1))


def _tc1_call(desc, tw, np8, cp8, w):
    out_shapes = (
        jax.ShapeDtypeStruct((NPAD, HID), F32),
        jax.ShapeDtypeStruct((NPAD, HID), F32),
        jax.ShapeDtypeStruct((NPAD, HID), F32),
        jax.ShapeDtypeStruct((NPAD, HID), F32),
        jax.ShapeDtypeStruct((NPAD, HID), F32),
        jax.ShapeDtypeStruct((NPAD, HID), F32),
        jax.ShapeDtypeStruct((1, HID), F32),
    )
    in_specs = [
        _rows((B, 768)), _rows((B, 768)), _rows((B, 8)), _rows((B, 8)),
        _full((768, 32)), _full((32,)), _full((768, 32)), _full((32,)),
        _full((8, 32)), _full((32,)), _full((8, 32)), _full((32,)),
        _full((4, 32, 32)), _full((4, 32)), _full((4, 32, 32)), _full((4, 32)),
        _full((64, 64)), _full((64,)), _full((64, 64)), _full((64,)),
        _full((64, 64)), _full((64,)), _full((64, 64)), _full((64,)),
        _full((64,)), _full((64,)),
        _full((64, HID)), _full((HID,)),
        _full((HID, HID)), _full((HID,)), _full((HID, HID)), _full((HID,)),
        _full((HID, HID)), _full((HID,)), _full((HID, HID)), _full((HID,)),
        _full((REL, HID)),
    ]
    out_specs = (
        _rows((B, HID)), _rows((B, HID)), _rows((B, HID)), _rows((B, HID)),
        _rows((B, HID)), _rows((B, HID)), _full((1, HID)),
    )
    return pl.pallas_call(
        _tc1_body, grid=(GRID,),
        in_specs=in_specs, out_specs=out_specs, out_shape=out_shapes,
    )(desc, tw, np8, cp8, *w)


def _tc2_call(accs, saccs, skip1, w):
    out_shapes = (
        jax.ShapeDtypeStruct((NPAD, HID), F32),
        jax.ShapeDtypeStruct((NPAD, HID), F32),
        jax.ShapeDtypeStruct((NPAD, HID), F32),
        jax.ShapeDtypeStruct((NPAD, HID), F32),
        jax.ShapeDtypeStruct((NPAD, HID), F32),
        jax.ShapeDtypeStruct((NPAD, HID), F32),
    )
    in_specs = [
        pl.BlockSpec((NC, B, HID), lambda i: (0, i, 0)),
        pl.BlockSpec((NC, REL, B, 16), lambda i: (0, 0, i, 0)),
        _rows((B, HID)),
        _full((REL, HID)),
        _full((HID, HID)), _full((HID,)), _full((HID, HID)), _full((HID,)),
        _full((HID, HID)), _full((HID,)), _full((HID, HID)), _full((HID,)),
        _full((REL, HID)),
    ]
    out_specs = (
        _rows((B, HID)), _rows((B, HID)), _rows((B, HID)), _rows((B, HID)),
        _rows((B, HID)), _rows((B, HID)),
    )
    return pl.pallas_call(
        _tc2_body, grid=(GRID,),
        in_specs=in_specs, out_specs=out_specs, out_shape=out_shapes,
    )(accs, saccs, skip1, *w)


def _tc3_call(accs, saccs, skip2, w):
    out_shapes = jax.ShapeDtypeStruct((NPAD, HID), F32)
    in_specs = [
        pl.BlockSpec((NC, B, HID), lambda i: (0, i, 0)),
        pl.BlockSpec((NC, REL, B, 16), lambda i: (0, 0, i, 0)),
        _rows((B, HID)),
        _full((REL, HID)),
        _full((HID, HID)), _full((HID,)), _full((HID, HID)), _full((HID,)),
    ]
    return pl.pallas_call(
        _tc3_body, grid=(GRID,),
        in_specs=in_specs, out_specs=(_rows((B, HID)),), out_shape=(out_shapes,),
    )(accs, saccs, skip2, *w)[0]


def _conv_weights(p, rel_emb, first):
    perm = PERM
    wq, wk, wv, ws = p['wq'], p['wk'], p['wv'], p['wskip']
    bq, bk, bv, bs = p['bq'], p['bk'], p['bv'], p['bskip']
    if not first:
        wq, wk, wv, ws = (w[perm, :] for w in (wq, wk, wv, ws))
    sc = 1.0 / np.sqrt(C)
    ek = (rel_emb @ p['we'])[:, perm]
    return [wq[:, perm] * sc, bq[perm] * sc,
            wk[:, perm], bk[perm],
            wv[:, perm], bv[perm],
            ws[:, perm], bs[perm],
            ek], ek


def kernel(description, tweet, num_prop, cat_prop, edge_index, edge_type, params):
    np8 = jnp.pad(num_prop, ((0, 0), (0, 3)))
    cp8 = jnp.pad(cat_prop, ((0, 0), (0, 5)))

    src = edge_index[0].astype(jnp.int32)
    dst = edge_index[1].astype(jnp.int32)
    et = edge_type.astype(jnp.int32)
    padn = EP - E
    padidx = N + (jnp.arange(padn, dtype=jnp.int32) % 16)
    srcp = jnp.concatenate([src, padidx])
    dstp = jnp.concatenate([dst, padidx])
    etp = jnp.concatenate([et, jnp.zeros((padn,), jnp.int32)])

    pm = params['mha']
    w1, ek1 = _conv_weights(params['conv1'], params['rel_emb'], True)
    w2, ek2 = _conv_weights(params['conv2'], params['rel_emb'], False)

    tc1_w = [
        params['desc']['w'], params['desc']['b'],
        params['tweet']['w'], params['tweet']['b'],
        jnp.pad(params['num']['w'], ((0, 3), (0, 0))), params['num']['b'],
        jnp.pad(params['cat']['w'], ((0, 5), (0, 0))), params['cat']['b'],
        jnp.stack([p['w'] for p in params['inv']]),
        jnp.stack([p['b'] for p in params['inv']]),
        jnp.stack([p['w'] for p in params['spec']]),
        jnp.stack([p['b'] for p in params['spec']]),
        pm['wq'], pm['bq'], pm['wk'], pm['bk'], pm['wv'], pm['bv'],
        pm['wo'], pm['bo'], params['ln_g'], params['ln_b'],
        params['c2h']['w'], params['c2h']['b'],
    ] + w1

    q1, k1, v1, s1, qe0, qe1, auxp = _tc1_call(description, tweet, np8, cp8, tc1_w)
    qe_t1 = jnp.concatenate([qe0, qe1], axis=0)

    accs1, saccs1 = _sc_edge(srcp, dstp, etp, q1, k1, v1, qe_t1)
    accs1 = accs1.reshape(NC, NPAD, HID)
    saccs1 = saccs1.reshape(NC, DR8P, HID)[:, :DR8].reshape(NC, REL, NPAD, 16)

    q2, k2, v2, s2, qe0b, qe1b = _tc2_call(accs1, saccs1, s1, [ek1] + w2)
    qe_t2 = jnp.concatenate([qe0b, qe1b], axis=0)

    accs2, saccs2 = _sc_edge(srcp, dstp, etp, q2, k2, v2, qe_t2)
    accs2 = accs2.reshape(NC, NPAD, HID)
    saccs2 = saccs2.reshape(NC, DR8P, HID)[:, :DR8].reshape(NC, REL, NPAD, 16)

    whp = jnp.pad(params['head']['w'], ((0, 0), (0, HID - 2)))
    bhp = jnp.pad(params['head']['b'], ((0, HID - 2),))
    tc3_w = [ek2, params['outmlp']['w'][PERM, :], params['outmlp']['b'], whp, bhp]
    lp = _tc3_call(accs2, saccs2, s2, tc3_w)

    logits = lp[:N, :2]
    inv_ss = auxp[0, 0]
    ov_ss = auxp[0, 1]
    aux = INV_W * (inv_ss / (N * 4 * 32) + 0.5 * ov_ss / (N * 6))
    return logits, aux
